# baseline (device time: 46469 ns/iter reference)
import jax
import jax.numpy as jnp
from jax import lax
from jax.experimental import pallas as pl
from jax.experimental.pallas import tpu as pltpu

N_DEV = 4
BLOCK_ROWS = 1024
SUB_ROWS = 16


def _totals_body(x_ref, t_ref):
    b = pl.program_id(0)
    blk = x_ref[...]
    r = blk.shape[0]
    while r > 1:
        h = r // 2
        blk = blk[:h, :] * blk[h:r, :]
        r = h

    @pl.when(b == 0)
    def _():
        t_ref[...] = blk

    @pl.when(b > 0)
    def _():
        t_ref[...] = t_ref[...] * blk


def _collective_body(t_ref, p_ref, comm_ref, send_sems, recv_sems):
    my = lax.axis_index("i")
    left = lax.rem(my + N_DEV - 1, N_DEV)
    right = lax.rem(my + 1, N_DEV)
    n = t_ref.shape[1]

    barrier_sem = pltpu.get_barrier_semaphore()
    for nbr in (left, right):
        pl.semaphore_signal(
            barrier_sem, inc=1,
            device_id=(nbr,), device_id_type=pl.DeviceIdType.MESH,
        )
    pl.semaphore_wait(barrier_sem, 2)

    comm_ref[0] = jnp.broadcast_to(t_ref[...], (8, n))

    p = jnp.ones((1, n), jnp.float32)
    for h in range(N_DEV - 1):
        rdma = pltpu.make_async_remote_copy(
            src_ref=comm_ref.at[h],
            dst_ref=comm_ref.at[h + 1],
            send_sem=send_sems.at[h],
            recv_sem=recv_sems.at[h],
            device_id=(right,),
            device_id_type=pl.DeviceIdType.MESH,
        )
        rdma.start()
        rdma.wait()
        chunk = comm_ref[h + 1, 0:1, :]
        p = p * jnp.where(my > h, chunk, jnp.ones_like(chunk))
    p_ref[...] = p


def _scan_fused_body(
    x_ref, t_ref, out_ref, carry_ref, comm_ref, send_sems, recv_sems,
    ack_sems, *, sub_rows,
):
    b = pl.program_id(0)
    nb = pl.num_programs(0)
    my = lax.axis_index("i")
    rows, n = x_ref.shape
    ones_row = jnp.ones((1, n), jnp.float32)

    @pl.when(b == 0)
    def _():
        barrier_sem = pltpu.get_barrier_semaphore()
        for delta in (1, 2, 3):
            pl.semaphore_signal(
                barrier_sem, inc=1,
                device_id=(lax.rem(my + delta, N_DEV),),
                device_id_type=pl.DeviceIdType.MESH,
            )
        pl.semaphore_wait(barrier_sem, N_DEV - 1)
        for d in (1, 2, 3):
            @pl.when(my + d <= N_DEV - 1)
            def _(d=d):
                pltpu.make_async_remote_copy(
                    src_ref=t_ref,
                    dst_ref=comm_ref.at[my],
                    send_sem=send_sems.at[d - 1],
                    recv_sem=recv_sems.at[my],
                    device_id=(my + d,),
                    device_id_type=pl.DeviceIdType.MESH,
                ).start()
    blk = x_ref[...]

    @pl.when(b == 0)
    def _():
        chunks = []
        carry = ones_row
        for s in range(0, rows, sub_rows):
            acc = _cumprod_chunk(blk[s : s + sub_rows, :]) * carry
            chunks.append(acc)
            carry = acc[sub_rows - 1 : sub_rows, :]
        p = ones_row
        for s in (0, 1, 2):
            @pl.when(s < my)
            def _(s=s):
                pltpu.make_async_remote_copy(
                    src_ref=t_ref,
                    dst_ref=comm_ref.at[s],
                    send_sem=send_sems.at[0],
                    recv_sem=recv_sems.at[s],
                    device_id=(0,),
                    device_id_type=pl.DeviceIdType.MESH,
                ).wait_recv()
            p = p * jnp.where(s < my, comm_ref[s, :, :], ones_row)
            @pl.when(s < my)
            def _(s=s):
                pl.semaphore_signal(
                    ack_sems.at[my], inc=1,
                    device_id=(s,), device_id_type=pl.DeviceIdType.MESH,
                )
        for i, acc in enumerate(chunks):
            out_ref[i * sub_rows : (i + 1) * sub_rows, :] = acc * p
        carry_ref[...] = carry * p

    @pl.when(b > 0)
    def _():
        carry = carry_ref[...]
        for s in range(0, rows, sub_rows):
            acc = _cumprod_chunk(blk[s : s + sub_rows, :]) * carry
            out_ref[s : s + sub_rows, :] = acc
            carry = acc[sub_rows - 1 : sub_rows, :]
        carry_ref[...] = carry

    @pl.when(b == nb - 1)
    def _():
        for d in (1, 2, 3):
            @pl.when(my + d <= N_DEV - 1)
            def _(d=d):
                pltpu.make_async_remote_copy(
                    src_ref=t_ref,
                    dst_ref=comm_ref.at[0],
                    send_sem=send_sems.at[d - 1],
                    recv_sem=recv_sems.at[0],
                    device_id=(0,),
                    device_id_type=pl.DeviceIdType.MESH,
                ).wait_send()
                pl.semaphore_wait(ack_sems.at[lax.rem(my + d, N_DEV)], 1)


def _scan_fused_call(x, totals, block_rows=None, sub_rows=None):
    import functools

    m, n = x.shape
    block_rows = block_rows or BLOCK_ROWS
    sub_rows = sub_rows or SUB_ROWS
    nb = m // block_rows
    return pl.pallas_call(
        functools.partial(_scan_fused_body, sub_rows=sub_rows),
        grid=(nb,),
        in_specs=[
            pl.BlockSpec((block_rows, n), lambda b: (b, 0),
                         memory_space=pltpu.VMEM),
            pl.BlockSpec((1, n), lambda b: (0, 0),
                         memory_space=pltpu.VMEM),
        ],
        out_specs=pl.BlockSpec((block_rows, n), lambda b: (b, 0),
                               memory_space=pltpu.VMEM),
        out_shape=jax.ShapeDtypeStruct((m, n), jnp.float32),
        scratch_shapes=[
            pltpu.VMEM((1, n), jnp.float32),
            pltpu.VMEM((N_DEV - 1, 1, n), jnp.float32),
            pltpu.SemaphoreType.DMA((N_DEV - 1,)),
            pltpu.SemaphoreType.DMA((N_DEV - 1,)),
            pltpu.SemaphoreType.REGULAR((N_DEV,)),
        ],
        compiler_params=pltpu.CompilerParams(
            dimension_semantics=("arbitrary",),
            collective_id=0,
            vmem_limit_bytes=100 * 1024 * 1024,
        ),
    )(x, totals)


def _cumprod_chunk(chunk):
    r, n = chunk.shape
    d = 1
    while d < r:
        shifted = jnp.concatenate(
            [jnp.ones((d, n), jnp.float32), chunk[: r - d, :]], axis=0
        )
        chunk = chunk * shifted
        d *= 2
    return chunk


def _scan_body(x_ref, p_ref, out_ref, carry_ref, *, sub_rows):
    b = pl.program_id(0)

    @pl.when(b == 0)
    def _():
        carry_ref[...] = p_ref[...]

    blk = x_ref[...]
    rows, n = blk.shape
    carry = carry_ref[...]
    for s in range(0, rows, sub_rows):
        acc = _cumprod_chunk(blk[s : s + sub_rows, :])
        acc = acc * carry
        out_ref[s : s + sub_rows, :] = acc
        carry = acc[sub_rows - 1 : sub_rows, :]
    carry_ref[...] = carry


def _totals_call(x, block_rows=None):
    m, n = x.shape
    block_rows = block_rows or BLOCK_ROWS
    nb = m // block_rows
    return pl.pallas_call(
        _totals_body,
        grid=(nb,),
        in_specs=[
            pl.BlockSpec((block_rows, n), lambda b: (b, 0),
                         memory_space=pltpu.VMEM),
        ],
        out_specs=pl.BlockSpec((1, n), lambda b: (0, 0),
                               memory_space=pltpu.VMEM),
        out_shape=jax.ShapeDtypeStruct((1, n), jnp.float32),
        compiler_params=pltpu.CompilerParams(
            dimension_semantics=("arbitrary",),
            vmem_limit_bytes=100 * 1024 * 1024,
        ),
    )(x)


def _collective_call(totals):
    n = totals.shape[1]
    return pl.pallas_call(
        _collective_body,
        in_specs=[pl.BlockSpec(memory_space=pltpu.VMEM)],
        out_specs=pl.BlockSpec(memory_space=pltpu.VMEM),
        out_shape=jax.ShapeDtypeStruct((1, n), jnp.float32),
        scratch_shapes=[
            pltpu.VMEM((N_DEV, 8, n), jnp.float32),
            pltpu.SemaphoreType.DMA((N_DEV - 1,)),
            pltpu.SemaphoreType.DMA((N_DEV - 1,)),
        ],
        compiler_params=pltpu.CompilerParams(collective_id=0),
    )(totals)


def _scan_call(x, prefix, block_rows=None, sub_rows=None):
    import functools

    m, n = x.shape
    block_rows = block_rows or BLOCK_ROWS
    sub_rows = sub_rows or SUB_ROWS
    nb = m // block_rows
    return pl.pallas_call(
        functools.partial(_scan_body, sub_rows=sub_rows),
        grid=(nb,),
        in_specs=[
            pl.BlockSpec((block_rows, n), lambda b: (b, 0),
                         memory_space=pltpu.VMEM),
            pl.BlockSpec((1, n), lambda b: (0, 0),
                         memory_space=pltpu.VMEM),
        ],
        out_specs=pl.BlockSpec((block_rows, n), lambda b: (b, 0),
                               memory_space=pltpu.VMEM),
        out_shape=jax.ShapeDtypeStruct((m, n), jnp.float32),
        scratch_shapes=[pltpu.VMEM((1, n), jnp.float32)],
        compiler_params=pltpu.CompilerParams(
            dimension_semantics=("arbitrary",),
            vmem_limit_bytes=100 * 1024 * 1024,
        ),
    )(x, prefix)


CHUNK_ROWS = 1024


def _mono_body(
    x_ref, out_ref, xbuf, tbuf, ostage, comm_ref,
    in_sems, out_sems, send_sems, recv_sems, ack_sems, *, sub_rows,
):
    my = lax.axis_index("i")
    m, n = x_ref.shape
    nch = m // CHUNK_ROWS
    ones_row = jnp.ones((1, n), jnp.float32)

    def in_copy(c):
        return pltpu.make_async_copy(
            x_ref.at[pl.ds(c * CHUNK_ROWS, CHUNK_ROWS), :],
            xbuf.at[pl.ds(c * CHUNK_ROWS, CHUNK_ROWS), :],
            in_sems.at[c],
        )

    def out_copy(c):
        return pltpu.make_async_copy(
            ostage.at[c % 2],
            out_ref.at[pl.ds(c * CHUNK_ROWS, CHUNK_ROWS), :],
            out_sems.at[c % 2],
        )

    for c in range(nch):
        in_copy(c).start()
    t = ones_row
    for c in range(nch):
        in_copy(c).wait()
        blk = xbuf[pl.ds(c * CHUNK_ROWS, CHUNK_ROWS), :]
        r = CHUNK_ROWS
        while r > 1:
            h = r // 2
            blk = blk[:h, :] * blk[h:r, :]
            r = h
        t = t * blk
    tbuf[...] = t

    barrier_sem = pltpu.get_barrier_semaphore()
    for delta in (1, 2, 3):
        pl.semaphore_signal(
            barrier_sem, inc=1,
            device_id=(lax.rem(my + delta, N_DEV),),
            device_id_type=pl.DeviceIdType.MESH,
        )
    pl.semaphore_wait(barrier_sem, N_DEV - 1)
    for d in (1, 2, 3):
        @pl.when(my + d <= N_DEV - 1)
        def _(d=d):
            pltpu.make_async_remote_copy(
                src_ref=tbuf,
                dst_ref=comm_ref.at[my],
                send_sem=send_sems.at[d - 1],
                recv_sem=recv_sems.at[my],
                device_id=(my + d,),
                device_id_type=pl.DeviceIdType.MESH,
            ).start()
    p = ones_row
    for s in (0, 1, 2):
        @pl.when(s < my)
        def _(s=s):
            pltpu.make_async_remote_copy(
                src_ref=tbuf,
                dst_ref=comm_ref.at[s],
                send_sem=send_sems.at[0],
                recv_sem=recv_sems.at[s],
                device_id=(0,),
                device_id_type=pl.DeviceIdType.MESH,
            ).wait_recv()
        p = p * jnp.where(s < my, comm_ref[s, :, :], ones_row)
        @pl.when(s < my)
        def _(s=s):
            pl.semaphore_signal(
                ack_sems.at[my], inc=1,
                device_id=(s,), device_id_type=pl.DeviceIdType.MESH,
            )

    carry = p
    for c in range(nch):
        if c >= 2:
            out_copy(c - 2).wait()
        for s in range(0, CHUNK_ROWS, sub_rows):
            acc = _cumprod_chunk(
                xbuf[pl.ds(c * CHUNK_ROWS + s, sub_rows), :]
            ) * carry
            ostage[c % 2, pl.ds(s, sub_rows), :] = acc
            carry = acc[sub_rows - 1 : sub_rows, :]
        out_copy(c).start()
    out_copy(nch - 2).wait()
    out_copy(nch - 1).wait()

    for d in (1, 2, 3):
        @pl.when(my + d <= N_DEV - 1)
        def _(d=d):
            pltpu.make_async_remote_copy(
                src_ref=tbuf,
                dst_ref=comm_ref.at[0],
                send_sem=send_sems.at[d - 1],
                recv_sem=recv_sems.at[0],
                device_id=(0,),
                device_id_type=pl.DeviceIdType.MESH,
            ).wait_send()
            pl.semaphore_wait(ack_sems.at[lax.rem(my + d, N_DEV)], 1)


def _mono_call(x, sub_rows=None):
    import functools

    m, n = x.shape
    sub_rows = sub_rows or SUB_ROWS
    return pl.pallas_call(
        functools.partial(_mono_body, sub_rows=sub_rows),
        in_specs=[pl.BlockSpec(memory_space=pl.ANY)],
        out_specs=pl.BlockSpec(memory_space=pl.ANY),
        out_shape=jax.ShapeDtypeStruct((m, n), jnp.float32),
        scratch_shapes=[
            pltpu.VMEM((m, n), jnp.float32),
            pltpu.VMEM((1, n), jnp.float32),
            pltpu.VMEM((2, CHUNK_ROWS, n), jnp.float32),
            pltpu.VMEM((N_DEV - 1, 1, n), jnp.float32),
            pltpu.SemaphoreType.DMA((m // CHUNK_ROWS,)),
            pltpu.SemaphoreType.DMA((2,)),
            pltpu.SemaphoreType.DMA((N_DEV - 1,)),
            pltpu.SemaphoreType.DMA((N_DEV - 1,)),
            pltpu.SemaphoreType.REGULAR((N_DEV,)),
        ],
        compiler_params=pltpu.CompilerParams(
            collective_id=0,
            vmem_limit_bytes=100 * 1024 * 1024,
        ),
    )(x)


def _collective_direct_body(
    t_ref, p_ref, comm_ref, send_sems, recv_sems, ack_sems
):
    my = lax.axis_index("i")
    n = t_ref.shape[1]
    ones_row = jnp.ones((1, n), jnp.float32)

    barrier_sem = pltpu.get_barrier_semaphore()
    for delta in (1, 2, 3):
        pl.semaphore_signal(
            barrier_sem, inc=1,
            device_id=(lax.rem(my + delta, N_DEV),),
            device_id_type=pl.DeviceIdType.MESH,
        )
    pl.semaphore_wait(barrier_sem, N_DEV - 1)

    for d in (1, 2, 3):
        @pl.when(my + d <= N_DEV - 1)
        def _(d=d):
            pltpu.make_async_remote_copy(
                src_ref=t_ref,
                dst_ref=comm_ref.at[my],
                send_sem=send_sems.at[d - 1],
                recv_sem=recv_sems.at[my],
                device_id=(my + d,),
                device_id_type=pl.DeviceIdType.MESH,
            ).start()

    p = ones_row
    for s in (0, 1, 2):
        @pl.when(s < my)
        def _(s=s):
            pltpu.make_async_remote_copy(
                src_ref=t_ref,
                dst_ref=comm_ref.at[s],
                send_sem=send_sems.at[0],
                recv_sem=recv_sems.at[s],
                device_id=(0,),
                device_id_type=pl.DeviceIdType.MESH,
            ).wait_recv()
        p = p * jnp.where(s < my, comm_ref[s, :, :], ones_row)
        @pl.when(s < my)
        def _(s=s):
            pl.semaphore_signal(
                ack_sems.at[my], inc=1,
                device_id=(s,), device_id_type=pl.DeviceIdType.MESH,
            )
    p_ref[...] = p

    for d in (1, 2, 3):
        @pl.when(my + d <= N_DEV - 1)
        def _(d=d):
            pltpu.make_async_remote_copy(
                src_ref=t_ref,
                dst_ref=comm_ref.at[0],
                send_sem=send_sems.at[d - 1],
                recv_sem=recv_sems.at[0],
                device_id=(0,),
                device_id_type=pl.DeviceIdType.MESH,
            ).wait_send()
            pl.semaphore_wait(ack_sems.at[lax.rem(my + d, N_DEV)], 1)


def _collective_direct_call(totals):
    n = totals.shape[1]
    return pl.pallas_call(
        _collective_direct_body,
        in_specs=[pl.BlockSpec(memory_space=pltpu.VMEM)],
        out_specs=pl.BlockSpec(memory_space=pltpu.VMEM),
        out_shape=jax.ShapeDtypeStruct((1, n), jnp.float32),
        scratch_shapes=[
            pltpu.VMEM((N_DEV - 1, 1, n), jnp.float32),
            pltpu.SemaphoreType.DMA((N_DEV - 1,)),
            pltpu.SemaphoreType.DMA((N_DEV - 1,)),
            pltpu.SemaphoreType.REGULAR((N_DEV,)),
        ],
        compiler_params=pltpu.CompilerParams(collective_id=0),
    )(totals)


def kernel(x):
    totals = _totals_call(x)
    prefix = _collective_direct_call(totals)
    return _scan_call(x, prefix)


# device time: 44296 ns/iter; 1.0491x vs baseline; 1.0491x over previous
import jax
import jax.numpy as jnp
from jax import lax
from jax.experimental import pallas as pl
from jax.experimental.pallas import tpu as pltpu

N_DEV = 4
BLOCK_ROWS = 1024
SUB_ROWS = 64


def _totals_body(x_ref, t_ref):
    b = pl.program_id(0)
    blk = x_ref[...]
    r = blk.shape[0]
    while r > 1:
        h = r // 2
        blk = blk[:h, :] * blk[h:r, :]
        r = h

    @pl.when(b == 0)
    def _():
        t_ref[...] = blk

    @pl.when(b > 0)
    def _():
        t_ref[...] = t_ref[...] * blk


def _collective_body(t_ref, p_ref, comm_ref, send_sems, recv_sems):
    my = lax.axis_index("i")
    left = lax.rem(my + N_DEV - 1, N_DEV)
    right = lax.rem(my + 1, N_DEV)
    n = t_ref.shape[1]

    barrier_sem = pltpu.get_barrier_semaphore()
    for nbr in (left, right):
        pl.semaphore_signal(
            barrier_sem, inc=1,
            device_id=(nbr,), device_id_type=pl.DeviceIdType.MESH,
        )
    pl.semaphore_wait(barrier_sem, 2)

    comm_ref[0] = jnp.broadcast_to(t_ref[...], (8, n))

    p = jnp.ones((1, n), jnp.float32)
    for h in range(N_DEV - 1):
        rdma = pltpu.make_async_remote_copy(
            src_ref=comm_ref.at[h],
            dst_ref=comm_ref.at[h + 1],
            send_sem=send_sems.at[h],
            recv_sem=recv_sems.at[h],
            device_id=(right,),
            device_id_type=pl.DeviceIdType.MESH,
        )
        rdma.start()
        rdma.wait()
        chunk = comm_ref[h + 1, 0:1, :]
        p = p * jnp.where(my > h, chunk, jnp.ones_like(chunk))
    p_ref[...] = p


def _scan_fused_body(
    x_ref, t_ref, out_ref, carry_ref, comm_ref, send_sems, recv_sems,
    ack_sems, *, sub_rows,
):
    b = pl.program_id(0)
    nb = pl.num_programs(0)
    my = lax.axis_index("i")
    rows, n = x_ref.shape
    ones_row = jnp.ones((1, n), jnp.float32)

    @pl.when(b == 0)
    def _():
        barrier_sem = pltpu.get_barrier_semaphore()
        for delta in (1, 2, 3):
            pl.semaphore_signal(
                barrier_sem, inc=1,
                device_id=(lax.rem(my + delta, N_DEV),),
                device_id_type=pl.DeviceIdType.MESH,
            )
        pl.semaphore_wait(barrier_sem, N_DEV - 1)
        for d in (1, 2, 3):
            @pl.when(my + d <= N_DEV - 1)
            def _(d=d):
                pltpu.make_async_remote_copy(
                    src_ref=t_ref,
                    dst_ref=comm_ref.at[my],
                    send_sem=send_sems.at[d - 1],
                    recv_sem=recv_sems.at[my],
                    device_id=(my + d,),
                    device_id_type=pl.DeviceIdType.MESH,
                ).start()
    blk = x_ref[...]

    @pl.when(b == 0)
    def _():
        chunks = []
        carry = ones_row
        for s in range(0, rows, sub_rows):
            acc = _cumprod_chunk(blk[s : s + sub_rows, :]) * carry
            chunks.append(acc)
            carry = acc[sub_rows - 1 : sub_rows, :]
        p = ones_row
        for s in (0, 1, 2):
            @pl.when(s < my)
            def _(s=s):
                pltpu.make_async_remote_copy(
                    src_ref=t_ref,
                    dst_ref=comm_ref.at[s],
                    send_sem=send_sems.at[0],
                    recv_sem=recv_sems.at[s],
                    device_id=(0,),
                    device_id_type=pl.DeviceIdType.MESH,
                ).wait_recv()
            p = p * jnp.where(s < my, comm_ref[s, :, :], ones_row)
            @pl.when(s < my)
            def _(s=s):
                pl.semaphore_signal(
                    ack_sems.at[my], inc=1,
                    device_id=(s,), device_id_type=pl.DeviceIdType.MESH,
                )
        for i, acc in enumerate(chunks):
            out_ref[i * sub_rows : (i + 1) * sub_rows, :] = acc * p
        carry_ref[...] = carry * p

    @pl.when(b > 0)
    def _():
        carry = carry_ref[...]
        for s in range(0, rows, sub_rows):
            acc = _cumprod_chunk(blk[s : s + sub_rows, :]) * carry
            out_ref[s : s + sub_rows, :] = acc
            carry = acc[sub_rows - 1 : sub_rows, :]
        carry_ref[...] = carry

    @pl.when(b == nb - 1)
    def _():
        for d in (1, 2, 3):
            @pl.when(my + d <= N_DEV - 1)
            def _(d=d):
                pltpu.make_async_remote_copy(
                    src_ref=t_ref,
                    dst_ref=comm_ref.at[0],
                    send_sem=send_sems.at[d - 1],
                    recv_sem=recv_sems.at[0],
                    device_id=(0,),
                    device_id_type=pl.DeviceIdType.MESH,
                ).wait_send()
                pl.semaphore_wait(ack_sems.at[lax.rem(my + d, N_DEV)], 1)


def _scan_fused_call(x, totals, block_rows=None, sub_rows=None):
    import functools

    m, n = x.shape
    block_rows = block_rows or BLOCK_ROWS
    sub_rows = sub_rows or SUB_ROWS
    nb = m // block_rows
    return pl.pallas_call(
        functools.partial(_scan_fused_body, sub_rows=sub_rows),
        grid=(nb,),
        in_specs=[
            pl.BlockSpec((block_rows, n), lambda b: (b, 0),
                         memory_space=pltpu.VMEM),
            pl.BlockSpec((1, n), lambda b: (0, 0),
                         memory_space=pltpu.VMEM),
        ],
        out_specs=pl.BlockSpec((block_rows, n), lambda b: (b, 0),
                               memory_space=pltpu.VMEM),
        out_shape=jax.ShapeDtypeStruct((m, n), jnp.float32),
        scratch_shapes=[
            pltpu.VMEM((1, n), jnp.float32),
            pltpu.VMEM((N_DEV - 1, 1, n), jnp.float32),
            pltpu.SemaphoreType.DMA((N_DEV - 1,)),
            pltpu.SemaphoreType.DMA((N_DEV - 1,)),
            pltpu.SemaphoreType.REGULAR((N_DEV,)),
        ],
        compiler_params=pltpu.CompilerParams(
            dimension_semantics=("arbitrary",),
            collective_id=0,
            vmem_limit_bytes=100 * 1024 * 1024,
        ),
    )(x, totals)


def _cumprod_chunk(chunk):
    r, n = chunk.shape
    d = 1
    while d < r:
        shifted = jnp.concatenate(
            [jnp.ones((d, n), jnp.float32), chunk[: r - d, :]], axis=0
        )
        chunk = chunk * shifted
        d *= 2
    return chunk


def _scan_body(x_ref, p_ref, out_ref, carry_ref, *, sub_rows):
    b = pl.program_id(0)

    @pl.when(b == 0)
    def _():
        carry_ref[...] = p_ref[...]

    blk = x_ref[...]
    rows, n = blk.shape
    carry = carry_ref[...]
    for s in range(0, rows, sub_rows):
        acc = _cumprod_chunk(blk[s : s + sub_rows, :])
        acc = acc * carry
        out_ref[s : s + sub_rows, :] = acc
        carry = acc[sub_rows - 1 : sub_rows, :]
    carry_ref[...] = carry


def _totals_call(x, block_rows=None):
    m, n = x.shape
    block_rows = block_rows or BLOCK_ROWS
    nb = m // block_rows
    return pl.pallas_call(
        _totals_body,
        grid=(nb,),
        in_specs=[
            pl.BlockSpec((block_rows, n), lambda b: (b, 0),
                         memory_space=pltpu.VMEM),
        ],
        out_specs=pl.BlockSpec((1, n), lambda b: (0, 0),
                               memory_space=pltpu.VMEM),
        out_shape=jax.ShapeDtypeStruct((1, n), jnp.float32),
        compiler_params=pltpu.CompilerParams(
            dimension_semantics=("arbitrary",),
            vmem_limit_bytes=100 * 1024 * 1024,
        ),
    )(x)


def _collective_call(totals):
    n = totals.shape[1]
    return pl.pallas_call(
        _collective_body,
        in_specs=[pl.BlockSpec(memory_space=pltpu.VMEM)],
        out_specs=pl.BlockSpec(memory_space=pltpu.VMEM),
        out_shape=jax.ShapeDtypeStruct((1, n), jnp.float32),
        scratch_shapes=[
            pltpu.VMEM((N_DEV, 8, n), jnp.float32),
            pltpu.SemaphoreType.DMA((N_DEV - 1,)),
            pltpu.SemaphoreType.DMA((N_DEV - 1,)),
        ],
        compiler_params=pltpu.CompilerParams(collective_id=0),
    )(totals)


def _scan_call(x, prefix, block_rows=None, sub_rows=None):
    import functools

    m, n = x.shape
    block_rows = block_rows or BLOCK_ROWS
    sub_rows = sub_rows or SUB_ROWS
    nb = m // block_rows
    return pl.pallas_call(
        functools.partial(_scan_body, sub_rows=sub_rows),
        grid=(nb,),
        in_specs=[
            pl.BlockSpec((block_rows, n), lambda b: (b, 0),
                         memory_space=pltpu.VMEM),
            pl.BlockSpec((1, n), lambda b: (0, 0),
                         memory_space=pltpu.VMEM),
        ],
        out_specs=pl.BlockSpec((block_rows, n), lambda b: (b, 0),
                               memory_space=pltpu.VMEM),
        out_shape=jax.ShapeDtypeStruct((m, n), jnp.float32),
        scratch_shapes=[pltpu.VMEM((1, n), jnp.float32)],
        compiler_params=pltpu.CompilerParams(
            dimension_semantics=("arbitrary",),
            vmem_limit_bytes=100 * 1024 * 1024,
        ),
    )(x, prefix)


CHUNK_ROWS = 1024


def _mono_body(
    x_ref, out_ref, xbuf, tbuf, ostage, comm_ref,
    in_sems, out_sems, send_sems, recv_sems, ack_sems, *, sub_rows,
):
    my = lax.axis_index("i")
    m, n = x_ref.shape
    nch = m // CHUNK_ROWS
    ones_row = jnp.ones((1, n), jnp.float32)

    def in_copy(c):
        return pltpu.make_async_copy(
            x_ref.at[pl.ds(c * CHUNK_ROWS, CHUNK_ROWS), :],
            xbuf.at[pl.ds(c * CHUNK_ROWS, CHUNK_ROWS), :],
            in_sems.at[c],
        )

    def out_copy(c):
        return pltpu.make_async_copy(
            ostage.at[c % 2],
            out_ref.at[pl.ds(c * CHUNK_ROWS, CHUNK_ROWS), :],
            out_sems.at[c % 2],
        )

    for c in range(nch):
        in_copy(c).start()
    t = ones_row
    for c in range(nch):
        in_copy(c).wait()
        blk = xbuf[pl.ds(c * CHUNK_ROWS, CHUNK_ROWS), :]
        r = CHUNK_ROWS
        while r > 1:
            h = r // 2
            blk = blk[:h, :] * blk[h:r, :]
            r = h
        t = t * blk
    tbuf[...] = t

    barrier_sem = pltpu.get_barrier_semaphore()
    for delta in (1, 2, 3):
        pl.semaphore_signal(
            barrier_sem, inc=1,
            device_id=(lax.rem(my + delta, N_DEV),),
            device_id_type=pl.DeviceIdType.MESH,
        )
    pl.semaphore_wait(barrier_sem, N_DEV - 1)
    for d in (1, 2, 3):
        @pl.when(my + d <= N_DEV - 1)
        def _(d=d):
            pltpu.make_async_remote_copy(
                src_ref=tbuf,
                dst_ref=comm_ref.at[my],
                send_sem=send_sems.at[d - 1],
                recv_sem=recv_sems.at[my],
                device_id=(my + d,),
                device_id_type=pl.DeviceIdType.MESH,
            ).start()
    p = ones_row
    for s in (0, 1, 2):
        @pl.when(s < my)
        def _(s=s):
            pltpu.make_async_remote_copy(
                src_ref=tbuf,
                dst_ref=comm_ref.at[s],
                send_sem=send_sems.at[0],
                recv_sem=recv_sems.at[s],
                device_id=(0,),
                device_id_type=pl.DeviceIdType.MESH,
            ).wait_recv()
        p = p * jnp.where(s < my, comm_ref[s, :, :], ones_row)
        @pl.when(s < my)
        def _(s=s):
            pl.semaphore_signal(
                ack_sems.at[my], inc=1,
                device_id=(s,), device_id_type=pl.DeviceIdType.MESH,
            )

    carry = p
    for c in range(nch):
        if c >= 2:
            out_copy(c - 2).wait()
        for s in range(0, CHUNK_ROWS, sub_rows):
            acc = _cumprod_chunk(
                xbuf[pl.ds(c * CHUNK_ROWS + s, sub_rows), :]
            ) * carry
            ostage[c % 2, pl.ds(s, sub_rows), :] = acc
            carry = acc[sub_rows - 1 : sub_rows, :]
        out_copy(c).start()
    out_copy(nch - 2).wait()
    out_copy(nch - 1).wait()

    for d in (1, 2, 3):
        @pl.when(my + d <= N_DEV - 1)
        def _(d=d):
            pltpu.make_async_remote_copy(
                src_ref=tbuf,
                dst_ref=comm_ref.at[0],
                send_sem=send_sems.at[d - 1],
                recv_sem=recv_sems.at[0],
                device_id=(0,),
                device_id_type=pl.DeviceIdType.MESH,
            ).wait_send()
            pl.semaphore_wait(ack_sems.at[lax.rem(my + d, N_DEV)], 1)


def _mono_call(x, sub_rows=None):
    import functools

    m, n = x.shape
    sub_rows = sub_rows or SUB_ROWS
    return pl.pallas_call(
        functools.partial(_mono_body, sub_rows=sub_rows),
        in_specs=[pl.BlockSpec(memory_space=pl.ANY)],
        out_specs=pl.BlockSpec(memory_space=pl.ANY),
        out_shape=jax.ShapeDtypeStruct((m, n), jnp.float32),
        scratch_shapes=[
            pltpu.VMEM((m, n), jnp.float32),
            pltpu.VMEM((1, n), jnp.float32),
            pltpu.VMEM((2, CHUNK_ROWS, n), jnp.float32),
            pltpu.VMEM((N_DEV - 1, 1, n), jnp.float32),
            pltpu.SemaphoreType.DMA((m // CHUNK_ROWS,)),
            pltpu.SemaphoreType.DMA((2,)),
            pltpu.SemaphoreType.DMA((N_DEV - 1,)),
            pltpu.SemaphoreType.DMA((N_DEV - 1,)),
            pltpu.SemaphoreType.REGULAR((N_DEV,)),
        ],
        compiler_params=pltpu.CompilerParams(
            collective_id=0,
            vmem_limit_bytes=100 * 1024 * 1024,
        ),
    )(x)


def _collective_direct_body(
    t_ref, p_ref, comm_ref, send_sems, recv_sems, ack_sems
):
    my = lax.axis_index("i")
    n = t_ref.shape[1]
    ones_row = jnp.ones((1, n), jnp.float32)

    barrier_sem = pltpu.get_barrier_semaphore()
    for delta in (1, 2, 3):
        pl.semaphore_signal(
            barrier_sem, inc=1,
            device_id=(lax.rem(my + delta, N_DEV),),
            device_id_type=pl.DeviceIdType.MESH,
        )
    pl.semaphore_wait(barrier_sem, N_DEV - 1)

    for d in (1, 2, 3):
        @pl.when(my + d <= N_DEV - 1)
        def _(d=d):
            pltpu.make_async_remote_copy(
                src_ref=t_ref,
                dst_ref=comm_ref.at[my],
                send_sem=send_sems.at[d - 1],
                recv_sem=recv_sems.at[my],
                device_id=(my + d,),
                device_id_type=pl.DeviceIdType.MESH,
            ).start()

    p = ones_row
    for s in (0, 1, 2):
        @pl.when(s < my)
        def _(s=s):
            pltpu.make_async_remote_copy(
                src_ref=t_ref,
                dst_ref=comm_ref.at[s],
                send_sem=send_sems.at[0],
                recv_sem=recv_sems.at[s],
                device_id=(0,),
                device_id_type=pl.DeviceIdType.MESH,
            ).wait_recv()
        p = p * jnp.where(s < my, comm_ref[s, :, :], ones_row)
        @pl.when(s < my)
        def _(s=s):
            pl.semaphore_signal(
                ack_sems.at[my], inc=1,
                device_id=(s,), device_id_type=pl.DeviceIdType.MESH,
            )
    p_ref[...] = p

    for d in (1, 2, 3):
        @pl.when(my + d <= N_DEV - 1)
        def _(d=d):
            pltpu.make_async_remote_copy(
                src_ref=t_ref,
                dst_ref=comm_ref.at[0],
                send_sem=send_sems.at[d - 1],
                recv_sem=recv_sems.at[0],
                device_id=(0,),
                device_id_type=pl.DeviceIdType.MESH,
            ).wait_send()
            pl.semaphore_wait(ack_sems.at[lax.rem(my + d, N_DEV)], 1)


def _collective_direct_call(totals):
    n = totals.shape[1]
    return pl.pallas_call(
        _collective_direct_body,
        in_specs=[pl.BlockSpec(memory_space=pltpu.VMEM)],
        out_specs=pl.BlockSpec(memory_space=pltpu.VMEM),
        out_shape=jax.ShapeDtypeStruct((1, n), jnp.float32),
        scratch_shapes=[
            pltpu.VMEM((N_DEV - 1, 1, n), jnp.float32),
            pltpu.SemaphoreType.DMA((N_DEV - 1,)),
            pltpu.SemaphoreType.DMA((N_DEV - 1,)),
            pltpu.SemaphoreType.REGULAR((N_DEV,)),
        ],
        compiler_params=pltpu.CompilerParams(collective_id=0),
    )(totals)


def kernel(x):
    totals = _totals_call(x)
    prefix = _collective_direct_call(totals)
    return _scan_call(x, prefix)
